# baseline (speedup 1.0000x reference)
import functools

import jax
import jax.numpy as jnp
from jax import lax
from jax.experimental import pallas as pl
from jax.experimental.pallas import tpu as pltpu
from jax.experimental.pallas import tpu_sc as plsc

NC = 2
NS = 16
NW = NC * NS
L = 16

CBA = 6

_SC_PARAMS = dict(
    compiler_params=pltpu.CompilerParams(
        needs_layout_passes=False, use_tc_tiling_on_sc=False),
)


def _sc_mesh():
    return plsc.VectorSubcoreMesh(core_axis_name="c", subcore_axis_name="s")


def _units(f):
    units = [(2 * i, 2) for i in range(f // 2)]
    if f % 2:
        units.append((f - 1, 1))
    return units


def _plan_passes(f):
    passes, cur, acc = [], [], 0
    for u in _units(f):
        if acc + u[1] > CBA:
            passes.append(cur)
            cur, acc = [], 0
        cur.append(u)
        acc += u[1]
    if cur:
        passes.append(cur)
    return passes


def _fill_iota(iota_v, nv):
    base = lax.iota(jnp.int32, L)
    for i in range(nv // L):
        iota_v[pl.ds(i * L, L)] = base + i * L
    if nv % L:
        iota_v[pl.ds(nv - L, L)] = base + (nv - L)


def _make_deg_kernel(n, e):
    ew = e // NW
    gw = ew // L
    nv = n // L

    @functools.partial(
        pl.kernel,
        out_type=[jax.ShapeDtypeStruct((nv, L), jnp.float32)] * NC,
        mesh=_sc_mesh(),
        scratch_types=[
            pltpu.VMEM((ew,), jnp.int32),
            pltpu.VMEM((nv, L), jnp.float32),
            pltpu.VMEM((nv,), jnp.int32),
            pltpu.VMEM_SHARED((nv, L), jnp.float32),
        ],
        **_SC_PARAMS,
    )
    def deg_kernel(ei_hbm, out0, out1, col_v, cnt_v, iota_v, shared):
        cid = lax.axis_index("c")
        sid = lax.axis_index("s")
        wid = sid * NC + cid
        base = wid * ew
        pltpu.sync_copy(ei_hbm.at[1, pl.ds(base, ew)], col_v)
        _fill_iota(iota_v, nv)

        zeros = jnp.zeros((L,), jnp.float32)

        @plsc.parallel_loop(0, nv)
        def _(i):
            cnt_v[i] = zeros

        @pl.when(sid == 0)
        def _():
            pltpu.sync_copy(cnt_v, shared)

        plsc.subcore_barrier()

        ones = jnp.ones((L,), jnp.float32)

        @plsc.parallel_loop(0, gw, unroll=4)
        def _(i):
            c = col_v[pl.ds(i * L, L)]
            plsc.addupdate_scatter(
                cnt_v, [jnp.right_shift(c, 4), jnp.bitwise_and(c, 15)], ones)

        pltpu.sync_copy(cnt_v, shared.at[iota_v], add=True)
        plsc.subcore_barrier()

        @pl.when(jnp.logical_and(sid == 0, cid == 0))
        def _():
            pltpu.sync_copy(shared, out0)

        @pl.when(jnp.logical_and(sid == 0, cid == 1))
        def _():
            pltpu.sync_copy(shared, out1)

    return deg_kernel


def _make_agg_kernel(n, e, f):
    ew = e // NW
    gw = ew // L
    nv = n // L
    passes = _plan_passes(f)
    ntab = len(_units(f))
    wtab = max(len(p) for p in passes)
    wacc = max(sum(u[1] for u in p) for p in passes)

    @functools.partial(
        pl.kernel,
        out_type=[jax.ShapeDtypeStruct((f * nv, L), jnp.float32)] * NC,
        mesh=_sc_mesh(),
        scratch_types=(
            [pltpu.VMEM((ew,), jnp.int32)] * 2
            + [pltpu.VMEM((nv,), jnp.int32)]
            + [pltpu.VMEM((n,), jnp.int32)] * wtab
            + [pltpu.VMEM((nv, L), jnp.float32)] * wacc
            + [pltpu.VMEM_SHARED((f, nv, L), jnp.float32)]
            + [pltpu.VMEM_SHARED((ntab, n), jnp.int32)]
        ),
        **_SC_PARAMS,
    )
    def agg_kernel(*refs):
        tab_hbms = refs[:ntab]
        ei_hbm, out0, out1 = refs[ntab:ntab + 3]
        row_v, col_v, iota_v = refs[ntab + 3:ntab + 6]
        tabs = refs[ntab + 6:ntab + 6 + wtab]
        accs = refs[ntab + 6 + wtab:ntab + 6 + wtab + wacc]
        shared = refs[ntab + 6 + wtab + wacc]
        stab = refs[ntab + 7 + wtab + wacc]

        cid = lax.axis_index("c")
        sid = lax.axis_index("s")
        wid = sid * NC + cid
        base = wid * ew
        pltpu.sync_copy(ei_hbm.at[0, pl.ds(base, ew)], row_v)
        pltpu.sync_copy(ei_hbm.at[1, pl.ds(base, ew)], col_v)
        _fill_iota(iota_v, nv)

        zeros = jnp.zeros((L,), jnp.float32)
        unit0 = 0
        for punits in passes:
            m = sum(u[1] for u in punits)
            feat0 = punits[0][0]

            @plsc.parallel_loop(0, nv)
            def _(i):
                for j in range(m):
                    accs[j][i] = zeros

            for t, (featu, _width) in enumerate(punits):
                @pl.when(sid == ((unit0 + t) % NS))
                def _():
                    pltpu.sync_copy(tab_hbms[unit0 + t], stab.at[unit0 + t])

            for j in range(m):
                @pl.when(sid == ((feat0 + j) % NS))
                def _():
                    pltpu.sync_copy(accs[j], shared.at[feat0 + j])

            plsc.subcore_barrier()

            for t in range(len(punits)):
                pltpu.sync_copy(stab.at[unit0 + t], tabs[t])

            @plsc.parallel_loop(0, gw, unroll=8)
            def _(i):
                r = row_v[pl.ds(i * L, L)]
                c = col_v[pl.ds(i * L, L)]
                chi = jnp.right_shift(c, 4)
                clo = jnp.bitwise_and(c, 15)
                j = 0
                for t, (featu, width) in enumerate(punits):
                    g = plsc.load_gather(tabs[t], [r])
                    if width == 2:
                        a, b = plsc.unpack(
                            plsc.bitcast(g, jnp.bfloat16),
                            format=plsc.PackFormat.INTERLEAVED,
                            preferred_element_type=jnp.float32)
                        plsc.addupdate_scatter(accs[j], [chi, clo], a)
                        plsc.addupdate_scatter(accs[j + 1], [chi, clo], b)
                    else:
                        plsc.addupdate_scatter(
                            accs[j], [chi, clo], plsc.bitcast(g, jnp.float32))
                    j += width

            for j in range(m):
                pltpu.sync_copy(accs[j], shared.at[feat0 + j].at[iota_v],
                                add=True)

            plsc.subcore_barrier()

            for j in range(m):
                feat = feat0 + j

                @pl.when(jnp.logical_and(sid == (feat % NS), cid == 0))
                def _():
                    pltpu.sync_copy(shared.at[feat],
                                    out0.at[pl.ds(feat * nv, nv)])

                @pl.when(jnp.logical_and(sid == (feat % NS), cid == 1))
                def _():
                    pltpu.sync_copy(shared.at[feat],
                                    out1.at[pl.ds(feat * nv, nv)])

            unit0 += len(punits)

    return agg_kernel


def _pack_cols(cols):
    outs = []
    for feat, width in _units(len(cols)):
        if width == 2:
            au = lax.convert_element_type(
                lax.bitcast_convert_type(
                    lax.convert_element_type(cols[feat], jnp.bfloat16),
                    jnp.uint16), jnp.uint32)
            bu = lax.convert_element_type(
                lax.bitcast_convert_type(
                    lax.convert_element_type(cols[feat + 1], jnp.bfloat16),
                    jnp.uint16), jnp.uint32)
            outs.append(lax.bitcast_convert_type(
                au | (bu << 16), jnp.int32))
        else:
            outs.append(lax.bitcast_convert_type(cols[feat], jnp.int32))
    return outs


def _make_tc1(n, d, h):
    nt = len(_units(h))

    def body(x_ref, w1_ref, deg0_ref, deg1_ref, *outs):
        xwT = lax.dot_general(
            w1_ref[...], x_ref[...], (((0,), (1,)), ((), ())),
            preferred_element_type=jnp.float32)
        deg = deg0_ref[...] + deg1_ref[...] + 1.0
        dis = lax.rsqrt(deg)
        cols = [dis * xwT[feat] for feat in range(h)]
        for t, p in enumerate(_pack_cols(cols)):
            outs[t][...] = p
        for feat in range(h):
            outs[nt + feat][...] = cols[feat]
        outs[nt + h][...] = dis

    out_shape = ([jax.ShapeDtypeStruct((n,), jnp.int32) for _ in range(nt)]
                 + [jax.ShapeDtypeStruct((n,), jnp.float32)
                    for _ in range(h + 1)])
    return pl.pallas_call(body, out_shape=out_shape)


def _make_tc2(n, h, c):
    nt = len(_units(c))

    def body(*refs):
        p0_ref, p1_ref = refs[0:2]
        xw_cols = refs[2:2 + h]
        dis_ref, b1_ref, w2_ref = refs[2 + h:5 + h]
        outs = refs[5 + h:]
        dis = dis_ref[...]
        psum = jnp.stack([p0_ref[pl.ds(feat * n, n)]
                          + p1_ref[pl.ds(feat * n, n)]
                          + xw_cols[feat][...] for feat in range(h)])
        a1 = dis * psum + b1_ref[...]
        hmat = jnp.maximum(a1, 0.0)
        xw2T = jnp.dot(w2_ref[...].T, hmat,
                       preferred_element_type=jnp.float32)
        cols = [dis * xw2T[feat] for feat in range(c)]
        for t, p in enumerate(_pack_cols(cols)):
            outs[t][...] = p
        for feat in range(c):
            outs[nt + feat][...] = cols[feat]

    out_shape = ([jax.ShapeDtypeStruct((n,), jnp.int32) for _ in range(nt)]
                 + [jax.ShapeDtypeStruct((n,), jnp.float32)
                    for _ in range(c)])
    return pl.pallas_call(body, out_shape=out_shape)


def _make_tc3(n, c):
    def body(*refs):
        q0_ref, q1_ref = refs[0:2]
        xw_cols = refs[2:2 + c]
        dis_ref, b2_ref, out_ref = refs[2 + c:]
        dis = dis_ref[...]
        qsum = jnp.stack([q0_ref[pl.ds(feat * n, n)]
                          + q1_ref[pl.ds(feat * n, n)]
                          + xw_cols[feat][...] for feat in range(c)])
        a2 = dis * qsum + b2_ref[...]
        m = jnp.max(a2, axis=0, keepdims=True)
        s = a2 - m
        lse = jnp.log(jnp.sum(jnp.exp(s), axis=0, keepdims=True))
        out_ref[...] = s - lse

    return pl.pallas_call(
        body, out_shape=jax.ShapeDtypeStruct((c, n), jnp.float32))


def kernel(x, edge_index, W1, b1, W2, b2):
    n, d = x.shape
    e = edge_index.shape[1]
    h = W1.shape[1]
    c = W2.shape[1]
    nt1 = len(_units(h))
    nt2 = len(_units(c))

    ei = edge_index.astype(jnp.int32)
    b1c = b1.reshape(h, 1)
    b2c = b2.reshape(c, 1)

    deg0, deg1 = _make_deg_kernel(n, e)(ei)

    tc1_outs = _make_tc1(n, d, h)(x, W1, deg0.reshape(n), deg1.reshape(n))
    tabs1 = tc1_outs[:nt1]
    xws1_cols = tc1_outs[nt1:nt1 + h]
    dis = tc1_outs[nt1 + h]

    p0, p1 = _make_agg_kernel(n, e, h)(*tabs1, ei)

    tc2_outs = _make_tc2(n, h, c)(
        p0.reshape(h * n), p1.reshape(h * n), *xws1_cols, dis, b1c, W2)
    tabs2 = tc2_outs[:nt2]
    xws2_cols = tc2_outs[nt2:]

    q0, q1 = _make_agg_kernel(n, e, c)(*tabs2, ei)

    outT = _make_tc3(n, c)(
        q0.reshape(c * n), q1.reshape(c * n), *xws2_cols, dis, b2c)
    return outT.T


# speedup vs baseline: 1.0228x; 1.0228x over previous
import functools

import jax
import jax.numpy as jnp
from jax import lax
from jax.experimental import pallas as pl
from jax.experimental.pallas import tpu as pltpu
from jax.experimental.pallas import tpu_sc as plsc

NC = 2
NS = 16
NW = NC * NS
L = 16

CBA = 6

_SC_PARAMS = dict(
    compiler_params=pltpu.CompilerParams(
        needs_layout_passes=False, use_tc_tiling_on_sc=False),
)


def _sc_mesh():
    return plsc.VectorSubcoreMesh(core_axis_name="c", subcore_axis_name="s")


def _units(f):
    units = [(2 * i, 2) for i in range(f // 2)]
    if f % 2:
        units.append((f - 1, 1))
    return units


def _plan_passes(f):
    passes, cur, acc = [], [], 0
    for u in _units(f):
        if acc + u[1] > CBA:
            passes.append(cur)
            cur, acc = [], 0
        cur.append(u)
        acc += u[1]
    if cur:
        passes.append(cur)
    return passes


def _fill_iota(iota_v, nv):
    base = lax.iota(jnp.int32, L)
    for i in range(nv // L):
        iota_v[pl.ds(i * L, L)] = base + i * L
    if nv % L:
        iota_v[pl.ds(nv - L, L)] = base + (nv - L)


def _make_deg_kernel(n, e):
    ew = e // NW
    gw = ew // L
    nv = n // L

    @functools.partial(
        pl.kernel,
        out_type=[jax.ShapeDtypeStruct((nv, L), jnp.float32)] * NC,
        mesh=_sc_mesh(),
        scratch_types=[
            pltpu.VMEM((ew,), jnp.int32),
            pltpu.VMEM((nv, L), jnp.float32),
            pltpu.VMEM((nv,), jnp.int32),
            pltpu.VMEM_SHARED((nv, L), jnp.float32),
        ],
        **_SC_PARAMS,
    )
    def deg_kernel(ei_hbm, out0, out1, col_v, cnt_v, iota_v, shared):
        cid = lax.axis_index("c")
        sid = lax.axis_index("s")
        wid = sid * NC + cid
        base = wid * ew
        pltpu.sync_copy(ei_hbm.at[1, pl.ds(base, ew)], col_v)
        _fill_iota(iota_v, nv)

        zeros = jnp.zeros((L,), jnp.float32)

        @plsc.parallel_loop(0, nv)
        def _(i):
            cnt_v[i] = zeros

        @pl.when(sid == 0)
        def _():
            pltpu.sync_copy(cnt_v, shared)

        plsc.subcore_barrier()

        ones = jnp.ones((L,), jnp.float32)

        @plsc.parallel_loop(0, gw, unroll=4)
        def _(i):
            c = col_v[pl.ds(i * L, L)]
            plsc.addupdate_scatter(
                cnt_v, [jnp.right_shift(c, 4), jnp.bitwise_and(c, 15)], ones)

        pltpu.sync_copy(cnt_v, shared.at[iota_v], add=True)
        plsc.subcore_barrier()

        @pl.when(jnp.logical_and(sid == 0, cid == 0))
        def _():
            pltpu.sync_copy(shared, out0)

        @pl.when(jnp.logical_and(sid == 0, cid == 1))
        def _():
            pltpu.sync_copy(shared, out1)

    return deg_kernel


def _make_agg_kernel(n, e, f):
    ew = e // NW
    gw = ew // L
    nv = n // L
    passes = _plan_passes(f)
    ntab = len(_units(f))
    wtab = max(len(p) for p in passes)
    wacc = max(sum(u[1] for u in p) for p in passes)

    @functools.partial(
        pl.kernel,
        out_type=[jax.ShapeDtypeStruct((f * nv, L), jnp.float32)] * NC,
        mesh=_sc_mesh(),
        scratch_types=(
            [pltpu.VMEM((ew,), jnp.int32)] * 2
            + [pltpu.VMEM((nv,), jnp.int32)]
            + [pltpu.VMEM((n,), jnp.int32)] * wtab
            + [pltpu.VMEM((nv, L), jnp.float32)] * wacc
            + [pltpu.VMEM_SHARED((f, nv, L), jnp.float32)]
            + [pltpu.VMEM_SHARED((ntab, n), jnp.int32)]
        ),
        **_SC_PARAMS,
    )
    def agg_kernel(*refs):
        tab_hbms = refs[:ntab]
        ei_hbm, out0, out1 = refs[ntab:ntab + 3]
        row_v, col_v, iota_v = refs[ntab + 3:ntab + 6]
        tabs = refs[ntab + 6:ntab + 6 + wtab]
        accs = refs[ntab + 6 + wtab:ntab + 6 + wtab + wacc]
        shared = refs[ntab + 6 + wtab + wacc]
        stab = refs[ntab + 7 + wtab + wacc]

        cid = lax.axis_index("c")
        sid = lax.axis_index("s")
        wid = sid * NC + cid
        base = wid * ew
        pltpu.sync_copy(ei_hbm.at[0, pl.ds(base, ew)], row_v)
        pltpu.sync_copy(ei_hbm.at[1, pl.ds(base, ew)], col_v)
        _fill_iota(iota_v, nv)

        zeros = jnp.zeros((L,), jnp.float32)
        unit0 = 0
        for punits in passes:
            m = sum(u[1] for u in punits)
            feat0 = punits[0][0]

            @plsc.parallel_loop(0, nv)
            def _(i):
                for j in range(m):
                    accs[j][i] = zeros

            for t, (featu, _width) in enumerate(punits):
                @pl.when(sid == ((unit0 + t) % NS))
                def _():
                    pltpu.sync_copy(tab_hbms[unit0 + t], stab.at[unit0 + t])

            for j in range(m):
                @pl.when(sid == ((feat0 + j) % NS))
                def _():
                    pltpu.sync_copy(accs[j], shared.at[feat0 + j])

            plsc.subcore_barrier()

            for t in range(len(punits)):
                pltpu.sync_copy(stab.at[unit0 + t], tabs[t])

            @plsc.parallel_loop(0, gw, unroll=4)
            def _(i):
                r = row_v[pl.ds(i * L, L)]
                c = col_v[pl.ds(i * L, L)]
                chi = jnp.right_shift(c, 4)
                clo = jnp.bitwise_and(c, 15)
                j = 0
                for t, (featu, width) in enumerate(punits):
                    g = plsc.load_gather(tabs[t], [r])
                    if width == 2:
                        a, b = plsc.unpack(
                            plsc.bitcast(g, jnp.bfloat16),
                            format=plsc.PackFormat.INTERLEAVED,
                            preferred_element_type=jnp.float32)
                        plsc.addupdate_scatter(accs[j], [chi, clo], a)
                        plsc.addupdate_scatter(accs[j + 1], [chi, clo], b)
                    else:
                        plsc.addupdate_scatter(
                            accs[j], [chi, clo], plsc.bitcast(g, jnp.float32))
                    j += width

            for j in range(m):
                pltpu.sync_copy(accs[j], shared.at[feat0 + j].at[iota_v],
                                add=True)

            plsc.subcore_barrier()

            for j in range(m):
                feat = feat0 + j

                @pl.when(jnp.logical_and(sid == (feat % NS), cid == 0))
                def _():
                    pltpu.sync_copy(shared.at[feat],
                                    out0.at[pl.ds(feat * nv, nv)])

                @pl.when(jnp.logical_and(sid == (feat % NS), cid == 1))
                def _():
                    pltpu.sync_copy(shared.at[feat],
                                    out1.at[pl.ds(feat * nv, nv)])

            unit0 += len(punits)

    return agg_kernel


def _pack_cols(cols):
    outs = []
    for feat, width in _units(len(cols)):
        if width == 2:
            au = lax.convert_element_type(
                lax.bitcast_convert_type(
                    lax.convert_element_type(cols[feat], jnp.bfloat16),
                    jnp.uint16), jnp.uint32)
            bu = lax.convert_element_type(
                lax.bitcast_convert_type(
                    lax.convert_element_type(cols[feat + 1], jnp.bfloat16),
                    jnp.uint16), jnp.uint32)
            outs.append(lax.bitcast_convert_type(
                au | (bu << 16), jnp.int32))
        else:
            outs.append(lax.bitcast_convert_type(cols[feat], jnp.int32))
    return outs


def _make_tc1a(n, d, h):
    def body(x_ref, w1_ref, *outs):
        xwT = lax.dot_general(
            w1_ref[...], x_ref[...], (((0,), (1,)), ((), ())),
            preferred_element_type=jnp.float32)
        for feat in range(h):
            outs[feat][...] = xwT[feat]

    out_shape = [jax.ShapeDtypeStruct((n,), jnp.float32) for _ in range(h)]
    return pl.pallas_call(body, out_shape=out_shape)


def _make_tc1b(n, h):
    nt = len(_units(h))

    def body(*refs):
        deg0_ref, deg1_ref = refs[0:2]
        xw_cols = refs[2:2 + h]
        outs = refs[2 + h:]
        deg = deg0_ref[...] + deg1_ref[...] + 1.0
        dis = lax.rsqrt(deg)
        cols = [dis * xw_cols[feat][...] for feat in range(h)]
        for t, p in enumerate(_pack_cols(cols)):
            outs[t][...] = p
        for feat in range(h):
            outs[nt + feat][...] = cols[feat]
        outs[nt + h][...] = dis

    out_shape = ([jax.ShapeDtypeStruct((n,), jnp.int32) for _ in range(nt)]
                 + [jax.ShapeDtypeStruct((n,), jnp.float32)
                    for _ in range(h + 1)])
    return pl.pallas_call(body, out_shape=out_shape)


def _make_tc2(n, h, c):
    nt = len(_units(c))

    def body(*refs):
        p0_ref, p1_ref = refs[0:2]
        xw_cols = refs[2:2 + h]
        dis_ref, b1_ref, w2_ref = refs[2 + h:5 + h]
        outs = refs[5 + h:]
        dis = dis_ref[...]
        psum = jnp.stack([p0_ref[pl.ds(feat * n, n)]
                          + p1_ref[pl.ds(feat * n, n)]
                          + xw_cols[feat][...] for feat in range(h)])
        a1 = dis * psum + b1_ref[...]
        hmat = jnp.maximum(a1, 0.0)
        xw2T = jnp.dot(w2_ref[...].T, hmat,
                       preferred_element_type=jnp.float32)
        cols = [dis * xw2T[feat] for feat in range(c)]
        for t, p in enumerate(_pack_cols(cols)):
            outs[t][...] = p
        for feat in range(c):
            outs[nt + feat][...] = cols[feat]

    out_shape = ([jax.ShapeDtypeStruct((n,), jnp.int32) for _ in range(nt)]
                 + [jax.ShapeDtypeStruct((n,), jnp.float32)
                    for _ in range(c)])
    return pl.pallas_call(body, out_shape=out_shape)


def _make_tc3(n, c):
    def body(*refs):
        q0_ref, q1_ref = refs[0:2]
        xw_cols = refs[2:2 + c]
        dis_ref, b2_ref, out_ref = refs[2 + c:]
        dis = dis_ref[...]
        qsum = jnp.stack([q0_ref[pl.ds(feat * n, n)]
                          + q1_ref[pl.ds(feat * n, n)]
                          + xw_cols[feat][...] for feat in range(c)])
        a2 = dis * qsum + b2_ref[...]
        m = jnp.max(a2, axis=0, keepdims=True)
        s = a2 - m
        lse = jnp.log(jnp.sum(jnp.exp(s), axis=0, keepdims=True))
        out_ref[...] = s - lse

    return pl.pallas_call(
        body, out_shape=jax.ShapeDtypeStruct((c, n), jnp.float32))


def kernel(x, edge_index, W1, b1, W2, b2):
    n, d = x.shape
    e = edge_index.shape[1]
    h = W1.shape[1]
    c = W2.shape[1]
    nt1 = len(_units(h))
    nt2 = len(_units(c))

    ei = edge_index.astype(jnp.int32)
    b1c = b1.reshape(h, 1)
    b2c = b2.reshape(c, 1)

    raw1_cols = _make_tc1a(n, d, h)(x, W1)
    deg0, deg1 = _make_deg_kernel(n, e)(ei)

    tc1_outs = _make_tc1b(n, h)(
        deg0.reshape(n), deg1.reshape(n), *raw1_cols)
    tabs1 = tc1_outs[:nt1]
    xws1_cols = tc1_outs[nt1:nt1 + h]
    dis = tc1_outs[nt1 + h]

    p0, p1 = _make_agg_kernel(n, e, h)(*tabs1, ei)

    tc2_outs = _make_tc2(n, h, c)(
        p0.reshape(h * n), p1.reshape(h * n), *xws1_cols, dis, b1c, W2)
    tabs2 = tc2_outs[:nt2]
    xws2_cols = tc2_outs[nt2:]

    q0, q1 = _make_agg_kernel(n, e, c)(*tabs2, ei)

    outT = _make_tc3(n, c)(
        q0.reshape(c * n), q1.reshape(c * n), *xws2_cols, dis, b2c)
    return outT.T
